# baseline (speedup 1.0000x reference)
import jax
import jax.numpy as jnp
from jax.experimental import pallas as pl
from jax.experimental.pallas import tpu as pltpu

_LANE = 128
_SUBLANE = 8
_ROW_TILE = 8192


def _round_up(x, m):
    return (x + m - 1) // m * m


def _enc_kernel(xt_ref, wp_ref, bias_ref, lat_ref, dims):
    C_pad, H, L, Q = dims
    t = pl.program_id(1)
    nt = pl.num_programs(1)
    xt = xt_ref[0]
    w1n = wp_ref[0:C_pad]
    w2t = wp_ref[C_pad:C_pad + L]
    TN = xt.shape[-1]
    CH = 2048 if TN % 2048 == 0 else TN
    zm = None
    for k in range(TN // CH):
        xk = xt[:, k * CH:(k + 1) * CH]
        htk = jax.lax.dot_general(w1n, xk, (((0,), (0,)), ((), ())),
                                  preferred_element_type=jnp.float32)
        hbk = jnp.maximum(htk.astype(jnp.bfloat16), 0)
        ztk = jnp.dot(w2t, hbk, preferred_element_type=jnp.float32)
        zmk = jnp.max(ztk, axis=1, keepdims=True)
        zm = zmk if zm is None else jnp.maximum(zm, zmk)

    @pl.when(t == 0)
    def _():
        lat_ref[...] = zm

    @pl.when(t > 0)
    def _():
        lat_ref[...] = jnp.maximum(lat_ref[...], zm)

    @pl.when(t == nt - 1)
    def _():
        r = C_pad + 2 * L + Q
        b2c = jnp.transpose(wp_ref[r + 1:r + 2, 0:L]).astype(jnp.float32)
        lat = jnp.maximum(lat_ref[...] + b2c, 0.0)
        lat_row = jnp.transpose(lat).astype(jnp.bfloat16)
        w3l = wp_ref[C_pad + L:C_pad + 2 * L]
        b3 = wp_ref[r:r + 1].astype(jnp.float32)
        bias_ref[0] = (jnp.dot(lat_row, w3l,
                               preferred_element_type=jnp.float32) + b3)


def _dec_kernel(x16_ref, w3qb_ref, wp_ref, o_ref, dims):
    C_pad, H, L, Q = dims
    x16 = x16_ref[0]
    ht = jnp.dot(w3qb_ref[0], x16,
                 preferred_element_type=jnp.float32)
    hb = jnp.maximum(ht.astype(jnp.bfloat16), 0)
    w4t = wp_ref[C_pad + 2 * L:C_pad + 2 * L + Q]
    r = C_pad + 2 * L + Q
    b4c = jnp.transpose(wp_ref[r + 2:r + 3, 0:Q]).astype(jnp.float32)
    ot = (jnp.dot(w4t, hb, preferred_element_type=jnp.float32)
          + b4c)
    o_ref[0] = jnp.transpose(ot)


def kernel(input_points, input_features, w1p, w1f, b1, w2, b2,
           w3q, w3l, b3, w4, b4):
    B, N, _ = input_points.shape
    D = input_features.shape[-1]
    H = w1p.shape[-1]
    L = w2.shape[-1]
    Q = w4.shape[-1]

    C = 4 + D
    C_pad = _round_up(C, 2 * _SUBLANE)
    H_pad = _round_up(H, _LANE)
    TN = min(_ROW_TILE, _round_up(N, _LANE))
    N_pad = _round_up(N, TN)
    dims = (C_pad, H, L, Q)

    x = jnp.concatenate(
        [input_points, jnp.ones((B, N, 1), jnp.float32), input_features,
         jnp.zeros((B, N, C_pad - C), jnp.float32)],
        axis=-1).astype(jnp.bfloat16)
    xt = jnp.transpose(x, (0, 2, 1))
    if N_pad != N:
        xt = jnp.pad(xt, ((0, 0), (0, 0), (0, N_pad - N)), mode="edge")

    bf16 = jnp.bfloat16
    rows = C_pad + 2 * L + Q + 3
    wpack = jnp.concatenate([
        w1p, b1, w1f, jnp.zeros((C_pad - C, H), jnp.float32),
        jnp.transpose(w2),
        w3l,
        jnp.transpose(w4),
        b3,
        jnp.pad(b2, ((0, 0), (0, H - L))),
        jnp.pad(b4, ((0, 0), (0, H - Q))),
    ], axis=0).astype(bf16)
    R_pad = _round_up(rows, 2 * _SUBLANE)
    if R_pad != rows:
        wpack = jnp.pad(wpack, ((0, R_pad - rows), (0, 0)))

    full = lambda shape: pl.BlockSpec(shape, lambda b, t: (0,) * len(shape))
    import functools
    bias = pl.pallas_call(
        functools.partial(_enc_kernel, dims=dims),
        out_shape=jax.ShapeDtypeStruct((B, 1, H), jnp.float32),
        grid=(B, N_pad // TN),
        in_specs=[
            pl.BlockSpec((1, C_pad, TN), lambda b, t: (b, 0, t)),
            full((R_pad, H)),
        ],
        out_specs=pl.BlockSpec((1, 1, H), lambda b, t: (b, 0, 0)),
        scratch_shapes=[pltpu.VMEM((L, 1), jnp.float32)],
        compiler_params=pltpu.CompilerParams(
            dimension_semantics=("parallel", "arbitrary")),
    )(xt, wpack)

    w3qt = jnp.transpose(w3q)
    w3qb = jnp.concatenate(
        [jnp.broadcast_to(w3qt[None], (B, H, 3)),
         jnp.transpose(bias, (0, 2, 1)),
         jnp.zeros((B, H, 12), jnp.float32)], axis=2).astype(bf16)

    out = pl.pallas_call(
        functools.partial(_dec_kernel, dims=dims),
        out_shape=jax.ShapeDtypeStruct((B, N_pad, Q), jnp.float32),
        grid=(B, N_pad // TN),
        in_specs=[
            pl.BlockSpec((1, 16, TN), lambda b, t: (b, 0, t)),
            pl.BlockSpec((1, H, 16), lambda b, t: (b, 0, 0)),
            full((R_pad, H)),
        ],
        out_specs=pl.BlockSpec((1, TN, Q), lambda b, t: (b, t, 0)),
        compiler_params=pltpu.CompilerParams(
            dimension_semantics=("parallel", "parallel")),
    )(xt, w3qb, wpack)

    if N_pad != N:
        return out[:, :N, :]
    return out


# speedup vs baseline: 1.0052x; 1.0052x over previous
import jax
import jax.numpy as jnp
from jax.experimental import pallas as pl
from jax.experimental.pallas import tpu as pltpu

_LANE = 128
_SUBLANE = 8
_ROW_TILE = 8192


def _round_up(x, m):
    return (x + m - 1) // m * m


def _enc_kernel(xt_ref, wp_ref, bias_ref, lat_ref, dims):
    C_pad, H, L, Q = dims
    t = pl.program_id(1)
    nt = pl.num_programs(1)
    xt = xt_ref[0]
    w1n = wp_ref[0:C_pad]
    w2t = wp_ref[C_pad:C_pad + L]
    ht = jax.lax.dot_general(w1n, xt, (((0,), (0,)), ((), ())),
                             preferred_element_type=jnp.float32)
    hb = jnp.maximum(ht.astype(jnp.bfloat16), 0)
    zt = jnp.dot(w2t, hb, preferred_element_type=jnp.float32)
    zm = jnp.max(zt, axis=1, keepdims=True)

    @pl.when(t == 0)
    def _():
        lat_ref[...] = zm

    @pl.when(t > 0)
    def _():
        lat_ref[...] = jnp.maximum(lat_ref[...], zm)

    @pl.when(t == nt - 1)
    def _():
        r = C_pad + 2 * L + Q
        b2c = jnp.transpose(wp_ref[r + 1:r + 2, 0:L]).astype(jnp.float32)
        lat = jnp.maximum(lat_ref[...] + b2c, 0.0)
        lat_row = jnp.transpose(lat).astype(jnp.bfloat16)
        w3l = wp_ref[C_pad + L:C_pad + 2 * L]
        b3 = wp_ref[r:r + 1].astype(jnp.float32)
        bias_ref[0] = (jnp.dot(lat_row, w3l,
                               preferred_element_type=jnp.float32) + b3)


def _dec_kernel(x16_ref, w3qb_ref, wp_ref, o_ref, dims):
    C_pad, H, L, Q = dims
    x16 = x16_ref[0]
    ht = jnp.dot(w3qb_ref[0], x16,
                 preferred_element_type=jnp.float32)
    hb = jnp.maximum(ht.astype(jnp.bfloat16), 0)
    w4t = wp_ref[C_pad + 2 * L:C_pad + 2 * L + Q]
    r = C_pad + 2 * L + Q
    b4c = jnp.transpose(wp_ref[r + 2:r + 3, 0:Q]).astype(jnp.float32)
    ot = (jnp.dot(w4t, hb, preferred_element_type=jnp.float32)
          + b4c)
    o_ref[0] = jnp.transpose(ot)


def kernel(input_points, input_features, w1p, w1f, b1, w2, b2,
           w3q, w3l, b3, w4, b4):
    B, N, _ = input_points.shape
    D = input_features.shape[-1]
    H = w1p.shape[-1]
    L = w2.shape[-1]
    Q = w4.shape[-1]

    C = 4 + D
    C_pad = _round_up(C, 2 * _SUBLANE)
    H_pad = _round_up(H, _LANE)
    TN = min(_ROW_TILE, _round_up(N, _LANE))
    N_pad = _round_up(N, TN)
    dims = (C_pad, H, L, Q)

    x = jnp.concatenate(
        [input_points, jnp.ones((B, N, 1), jnp.float32), input_features,
         jnp.zeros((B, N, C_pad - C), jnp.float32)],
        axis=-1).astype(jnp.bfloat16)
    xt = jnp.transpose(x, (0, 2, 1))
    if N_pad != N:
        xt = jnp.pad(xt, ((0, 0), (0, 0), (0, N_pad - N)), mode="edge")

    bf16 = jnp.bfloat16
    rows = C_pad + 2 * L + Q + 3
    wpack = jnp.concatenate([
        w1p, b1, w1f, jnp.zeros((C_pad - C, H), jnp.float32),
        jnp.transpose(w2),
        w3l,
        jnp.transpose(w4),
        b3,
        jnp.pad(b2, ((0, 0), (0, H - L))),
        jnp.pad(b4, ((0, 0), (0, H - Q))),
    ], axis=0).astype(bf16)
    R_pad = _round_up(rows, 2 * _SUBLANE)
    if R_pad != rows:
        wpack = jnp.pad(wpack, ((0, R_pad - rows), (0, 0)))

    full = lambda shape: pl.BlockSpec(shape, lambda b, t: (0,) * len(shape))
    import functools
    bias = pl.pallas_call(
        functools.partial(_enc_kernel, dims=dims),
        out_shape=jax.ShapeDtypeStruct((B, 1, H), jnp.float32),
        grid=(B, N_pad // TN),
        in_specs=[
            pl.BlockSpec((1, C_pad, TN), lambda b, t: (b, 0, t)),
            full((R_pad, H)),
        ],
        out_specs=pl.BlockSpec((1, 1, H), lambda b, t: (b, 0, 0)),
        scratch_shapes=[pltpu.VMEM((L, 1), jnp.float32)],
        compiler_params=pltpu.CompilerParams(
            dimension_semantics=("parallel", "arbitrary")),
    )(xt, wpack)

    w3qt = jnp.transpose(w3q)
    w3qb = jnp.concatenate(
        [jnp.broadcast_to(w3qt[None], (B, H, 3)),
         jnp.transpose(bias, (0, 2, 1)),
         jnp.zeros((B, H, 12), jnp.float32)], axis=2).astype(bf16)

    out = pl.pallas_call(
        functools.partial(_dec_kernel, dims=dims),
        out_shape=jax.ShapeDtypeStruct((B, N_pad, Q), jnp.float32),
        grid=(B, N_pad // TN),
        in_specs=[
            pl.BlockSpec((1, 16, TN), lambda b, t: (b, 0, t)),
            pl.BlockSpec((1, H, 16), lambda b, t: (b, 0, 0)),
            full((R_pad, H)),
        ],
        out_specs=pl.BlockSpec((1, TN, Q), lambda b, t: (b, t, 0)),
        compiler_params=pltpu.CompilerParams(
            dimension_semantics=("parallel", "parallel")),
    )(xt, w3qb, wpack)

    if N_pad != N:
        return out[:, :N, :]
    return out
